# baseline (speedup 1.0000x reference)
import functools

import jax
import jax.numpy as jnp
from jax import lax
from jax.experimental import pallas as pl
from jax.experimental.pallas import tpu as pltpu
from jax.experimental.pallas import tpu_sc as plsc

_NC, _NS, _L = 2, 16, 16
_NW = _NC * _NS
_B, _F = 16384, 26
_R = _B // _NW


def _body(idxT_hbm, w_hbm, out_hbm, *s):
    idxs = s[0:_F]
    vals = s[_F:2 * _F]
    out_v = s[2 * _F]
    sem = s[2 * _F + 1]
    sem_idx = s[2 * _F + 2]
    wid = lax.axis_index("s") * _NC + lax.axis_index("c")
    col0 = pl.multiple_of(wid * _R, _R)
    icps = [
        pltpu.async_copy(idxT_hbm.at[f, pl.ds(col0, _R)], idxs[f], sem_idx)
        for f in range(_F)
    ]
    gcps = []
    for f in range(_F):
        icps[f].wait()
        gcps.append(pltpu.async_copy(w_hbm.at[idxs[f]], vals[f], sem))

    _H = _F // 2
    for c in gcps[:_H]:
        c.wait()

    def red1(j, c):
        base = pl.multiple_of(j * _L, _L)
        acc = vals[0][pl.ds(base, _L)]
        for f in range(1, _H):
            acc = acc + vals[f][pl.ds(base, _L)]
        out_v[pl.ds(base, _L)] = acc
        return c

    lax.fori_loop(0, _R // _L, red1, 0)

    for c in gcps[_H:]:
        c.wait()

    def red2(j, c):
        base = pl.multiple_of(j * _L, _L)
        acc = out_v[pl.ds(base, _L)]
        for f in range(_H, _F):
            acc = acc + vals[f][pl.ds(base, _L)]
        out_v[pl.ds(base, _L)] = acc
        return c

    lax.fori_loop(0, _R // _L, red2, 0)
    pltpu.sync_copy(out_v, out_hbm.at[pl.ds(col0, _R)])


_sc_call = pl.kernel(
    _body,
    out_type=jax.ShapeDtypeStruct((_B,), jnp.float32),
    mesh=plsc.VectorSubcoreMesh(
        core_axis_name="c", subcore_axis_name="s",
        num_cores=_NC, num_subcores=_NS,
    ),
    scratch_types=(
        [pltpu.VMEM((_R,), jnp.int32) for _ in range(_F)]
        + [pltpu.VMEM((_R,), jnp.float32) for _ in range(_F)]
        + [pltpu.VMEM((_R,), jnp.float32),
           pltpu.SemaphoreType.DMA, pltpu.SemaphoreType.DMA]
    ),
    compiler_params=pltpu.CompilerParams(needs_layout_passes=False),
)


@jax.jit
def kernel(inputs, w):
    idxT = inputs.astype(jnp.int32).T
    return _sc_call(idxT, w.reshape(-1)).reshape(_B, 1)


# speedup vs baseline: 1.8078x; 1.8078x over previous
import functools

import jax
import jax.numpy as jnp
from jax import lax
from jax.experimental import pallas as pl
from jax.experimental.pallas import tpu as pltpu
from jax.experimental.pallas import tpu_sc as plsc

_NC, _NS, _L = 2, 16, 16
_NW = _NC * _NS
_B, _F = 16384, 26
_R = _B // _NW


def _body(idxT_hbm, w_hbm, out_hbm, *s):
    idxs = s[0:_F]
    vals = s[_F:2 * _F]
    out_v = s[2 * _F]
    sem = s[2 * _F + 1]
    sem_idx = s[2 * _F + 2]
    wid = lax.axis_index("s") * _NC + lax.axis_index("c")
    col0 = pl.multiple_of(wid * _R, _R)
    icps = [
        pltpu.async_copy(idxT_hbm.at[f, pl.ds(col0, _R)], idxs[f], sem_idx)
        for f in range(_F)
    ]
    gcps = []
    for f in range(_F):
        icps[f].wait()
        gcps.append(pltpu.async_copy(w_hbm.at[idxs[f]], vals[f], sem))

    _H = _F // 2
    for c in gcps[:_H]:
        c.wait()

    def red1(j, c):
        base = pl.multiple_of(j * _L, _L)
        acc = vals[0][pl.ds(base, _L)]
        for f in range(1, _H):
            acc = acc + vals[f][pl.ds(base, _L)]
        out_v[pl.ds(base, _L)] = acc
        return c

    lax.fori_loop(0, _R // _L, red1, 0)

    for c in gcps[_H:]:
        c.wait()

    def red2(j, c):
        base = pl.multiple_of(j * _L, _L)
        acc = out_v[pl.ds(base, _L)]
        for f in range(_H, _F):
            acc = acc + vals[f][pl.ds(base, _L)]
        out_v[pl.ds(base, _L)] = acc
        return c

    lax.fori_loop(0, _R // _L, red2, 0)
    pltpu.sync_copy(out_v, out_hbm.at[pl.ds(col0, _R)])


_sc_call = pl.kernel(
    _body,
    out_type=jax.ShapeDtypeStruct((_B,), jnp.float32),
    mesh=plsc.VectorSubcoreMesh(
        core_axis_name="c", subcore_axis_name="s",
        num_cores=_NC, num_subcores=_NS,
    ),
    scratch_types=(
        [pltpu.VMEM((_R,), jnp.int32) for _ in range(_F)]
        + [pltpu.VMEM((_R,), jnp.float32) for _ in range(_F)]
        + [pltpu.VMEM((_R,), jnp.float32),
           pltpu.SemaphoreType.DMA, pltpu.SemaphoreType.DMA]
    ),
    compiler_params=pltpu.CompilerParams(needs_layout_passes=False),
)


@jax.jit
def kernel(inputs, w):
    idxT = inputs.astype(jnp.int32).T
    w_pad = jnp.pad(w.T, ((0, 0), (0, 448))).reshape(-1)
    return _sc_call(idxT, w_pad).reshape(_B, 1)
